# baseline (speedup 1.0000x reference)
import functools

import jax
import jax.numpy as jnp
from jax import lax
from jax.experimental import pallas as pl
from jax.experimental.pallas import tpu as pltpu
from jax.experimental.pallas import tpu_sc as plsc

B = 16384
L = 4
V = 4
D = 8
TD = 64
LV = L * V

CH = 128


def _sc_gather(table2d, idx3d, n_ch, b_per_w, nc):
    mesh = plsc.VectorSubcoreMesh(core_axis_name="c", subcore_axis_name="s")

    @functools.partial(
        pl.kernel,
        mesh=mesh,
        out_type=jax.ShapeDtypeStruct((B, LV), jnp.float32),
        scratch_types=[
            pltpu.VMEM((n_ch, CH), jnp.int32),
            pltpu.VMEM((b_per_w, LV), jnp.float32),
            pltpu.SemaphoreType.DMA,
        ],
        compiler_params=pltpu.CompilerParams(use_tc_tiling_on_sc=False),
    )
    def k(table_hbm, idx_hbm, out_hbm, idx_v, rows_v, sem):
        wid = lax.axis_index("s") * nc + lax.axis_index("c")
        base = wid * b_per_w
        pltpu.sync_copy(idx_hbm.at[wid], idx_v)
        copies = []
        for j in range(n_ch):
            copies.append(
                pltpu.async_copy(
                    table_hbm.at[idx_v.at[j]],
                    rows_v.at[pl.ds(j * CH, CH)],
                    sem,
                )
            )
        for c in copies:
            c.wait()
        pltpu.sync_copy(rows_v, out_hbm.at[pl.ds(base, b_per_w)])

    return k(table2d, idx3d)


def _dense_body(g_ref, vr_ref, t_ref, wt_ref, b_ref, o_ref):
    x = g_ref[...] * t_ref[...]
    m = jnp.max(x, axis=1, keepdims=True)
    e = jnp.exp(x - m)
    ii = lax.broadcasted_iota(jnp.int32, (LV, LV), 0) // V
    jj = lax.broadcasted_iota(jnp.int32, (LV, LV), 1) // V
    gm = (ii == jj).astype(jnp.float32)
    s = jax.lax.dot(e, gm, preferred_element_type=jnp.float32,
                    precision=jax.lax.Precision.HIGHEST)
    w = e / s
    vr = vr_ref[...]
    vt = jnp.concatenate([vr, vr, vr, vr], axis=1)
    ri = lax.broadcasted_iota(jnp.int32, (LV, L * D), 0) // V
    ci = lax.broadcasted_iota(jnp.int32, (LV, L * D), 1) // D
    bd = jnp.where(ri == ci, vt, 0.0)
    mm = jax.lax.dot(bd, wt_ref[...], preferred_element_type=jnp.float32,
                     precision=jax.lax.Precision.HIGHEST)
    o_ref[...] = (
        jax.lax.dot(w, mm, preferred_element_type=jnp.float32,
                    precision=jax.lax.Precision.HIGHEST) + b_ref[...]
    )


def _tc_dense(g, vr, t_full, wt, b2d):
    blk = 2048
    grid = (B // blk,)
    return pl.pallas_call(
        _dense_body,
        grid=grid,
        in_specs=[
            pl.BlockSpec((blk, LV), lambda i: (i, 0)),
            pl.BlockSpec((LV, D), lambda i: (0, 0)),
            pl.BlockSpec((1, LV), lambda i: (0, 0)),
            pl.BlockSpec((L * D, TD), lambda i: (0, 0)),
            pl.BlockSpec((1, TD), lambda i: (0, 0)),
        ],
        out_specs=pl.BlockSpec((blk, TD), lambda i: (i, 0)),
        out_shape=jax.ShapeDtypeStruct((B, TD), jnp.float32),
    )(g, vr, t_full, wt, b2d)


def kernel(idx_batch, logits_table, vertices, logit_temps, W, b):
    info = plsc.get_sparse_core_info()
    nw = info.num_cores * info.num_subcores
    b_per_w = B // nw
    n_ch = b_per_w // CH

    table2d = logits_table.reshape(logits_table.shape[0], LV)
    idx3d = idx_batch.astype(jnp.int32).reshape(nw, n_ch, CH)
    g = _sc_gather(table2d, idx3d, n_ch, b_per_w, info.num_cores)

    vr = vertices.reshape(LV, D)
    t_full = jnp.repeat(logit_temps, V).reshape(1, LV)
    wt = W.T
    b2d = b.reshape(1, TD)
    return _tc_dense(g, vr, t_full, wt, b2d)
